# baseline (speedup 1.0000x reference)
import functools

import jax
import jax.numpy as jnp
from jax import lax
from jax.experimental import pallas as pl
from jax.experimental.pallas import tpu as pltpu
from jax.experimental.pallas import tpu_sc as plsc

B = 1024
F = 26
V = 2600
NW = 32
RPW = B // NW
CHUNK = RPW * V


def _encode_body(inp_hbm, oh_hbm, out_hbm, idx_v, oh_v, buf_v):
    wid = lax.axis_index("s") * 2 + lax.axis_index("c")
    base = wid * RPW

    pltpu.sync_copy(inp_hbm.at[pl.ds(base, RPW)], idx_v)
    pltpu.sync_copy(oh_hbm, oh_v)

    zeros = jnp.zeros((16,), jnp.int32)

    def zbody(i, carry):
        for j in range(8):
            buf_v[pl.ds(i * 128 + j * 16, 16)] = zeros
        return carry

    lax.fori_loop(0, CHUNK // 128, zbody, 0)

    ones = jnp.ones((16,), jnp.int32)
    oh_lo = oh_v[pl.ds(0, 16)]
    oh_hi = oh_v[pl.ds(F - 16, 16)]
    for r in range(RPW):
        pos_lo = idx_v[r, pl.ds(0, 16)] + oh_lo + (r * V)
        pos_hi = idx_v[r, pl.ds(F - 16, 16)] + oh_hi + (r * V)
        plsc.store_scatter(buf_v, [pos_lo], ones)
        plsc.store_scatter(buf_v, [pos_hi], ones)

    pltpu.sync_copy(buf_v, out_hbm.at[pl.ds(wid * CHUNK, CHUNK)])


_encode = functools.partial(
    pl.kernel,
    out_type=jax.ShapeDtypeStruct((B * V,), jnp.int32),
    mesh=plsc.VectorSubcoreMesh(core_axis_name="c", subcore_axis_name="s"),
    compiler_params=pltpu.CompilerParams(needs_layout_passes=False),
    scratch_types=[
        pltpu.VMEM((RPW, F), jnp.int32),
        pltpu.VMEM((F,), jnp.int32),
        pltpu.VMEM((CHUNK,), jnp.int32),
    ],
)(_encode_body)


def kernel(inputs, oh_indices):
    return _encode(inputs, oh_indices).reshape(B, V)
